# Initial kernel scaffold; baseline (speedup 1.0000x reference)
import jax
import jax.numpy as jnp
from jax.experimental import pallas as pl


def kernel(conditions_hash, procedures_hash, dx_table, proc_table, visit_table, gamma, beta):
    raise NotImplementedError("write your pallas kernel here")


# speedup vs baseline: 2.7787x; 2.7787x over previous
import functools

import jax
import jax.numpy as jnp
from jax import lax
from jax.experimental import pallas as pl
from jax.experimental.pallas import tpu as pltpu
from jax.experimental.pallas import tpu_sc as plsc

VOCAB = 100000
D = 64
B = 4096
L = 200

NC = 2
NS = 16
NW = NC * NS

N_ROWS = B * L
ROWS_PER_W = N_ROWS // NW
CHUNK = 512
GATHER = 128
K = CHUNK // GATHER
N_CHUNKS = ROWS_PER_W // CHUNK
IDX_ROWS = N_ROWS // GATHER
VIS_PER_W = B // NW


def _rsqrt(v):
    i = lax.bitcast_convert_type(v, jnp.int32)
    i = jnp.int32(0x5F3759DF) - lax.shift_right_logical(i, 1)
    y = lax.bitcast_convert_type(i, jnp.float32)
    h = v * jnp.float32(0.5)
    for _ in range(3):
        y = y * (jnp.float32(1.5) - h * y * y)
    return y


def _hsum(v):
    idx = lax.iota(jnp.int32, 16)
    for k in (1, 2, 4, 8):
        v = v + v.at[idx ^ k].get(mode="promise_in_bounds")
    return v


def _ln_row(xs, gs, bs):
    s = (xs[0] + xs[1]) + (xs[2] + xs[3])
    mean = _hsum(s) * jnp.float32(1.0 / D)
    sq = (xs[0] * xs[0] + xs[1] * xs[1]) + (xs[2] * xs[2] + xs[3] * xs[3])
    ex2 = _hsum(sq) * jnp.float32(1.0 / D)
    var = ex2 - mean * mean
    r = _rsqrt(var + jnp.float32(1e-5))
    return [(xs[j] - mean) * r * gs[j] + bs[j] for j in range(4)]


def _sc_kernel(cond_i, proc_i, dx_t, proc_t, visit_t, gamma, beta,
               cond_o, proc_o, visit_o,
               idx_v, rows_v, g_v, b_v, vis_v, sem):
    wid = lax.axis_index("s") * NC + lax.axis_index("c")

    pltpu.sync_copy(gamma, g_v)
    pltpu.sync_copy(beta, b_v)
    pltpu.sync_copy(visit_t, vis_v)

    gs = [g_v[pl.ds(16 * j, 16)] for j in range(4)]
    bs = [b_v[pl.ds(16 * j, 16)] for j in range(4)]

    vxs = [vis_v[0, pl.ds(16 * j, 16)] for j in range(4)]
    vout = _ln_row(vxs, gs, bs)

    @plsc.parallel_loop(0, VIS_PER_W, unroll=4)
    def _(r):
        for j in range(4):
            rows_v[r, pl.ds(16 * j, 16)] = vout[j]

    pltpu.sync_copy(rows_v.at[pl.ds(0, VIS_PER_W)],
                    visit_o.at[pl.ds(wid * VIS_PER_W, VIS_PER_W)])

    for idx_hbm, tab, out in ((cond_i, dx_t, cond_o), (proc_i, proc_t, proc_o)):
        def chunk_body(k, _, idx_hbm=idx_hbm, tab=tab, out=out):
            idx_base = wid * (ROWS_PER_W // GATHER) + k * K
            row_base = wid * ROWS_PER_W + k * CHUNK
            pltpu.sync_copy(idx_hbm.at[pl.ds(idx_base, K)], idx_v)
            cps = [
                pltpu.async_copy(tab.at[idx_v.at[j]],
                                 rows_v.at[pl.ds(j * GATHER, GATHER)], sem)
                for j in range(K)
            ]
            for cp in cps:
                cp.wait()

            @plsc.parallel_loop(0, CHUNK, unroll=8)
            def _(r):
                xs = [rows_v[r, pl.ds(16 * j, 16)] for j in range(4)]
                ys = _ln_row(xs, gs, bs)
                for j in range(4):
                    rows_v[r, pl.ds(16 * j, 16)] = ys[j]

            pltpu.sync_copy(rows_v, out.at[pl.ds(row_base, CHUNK)])
            return _

        lax.fori_loop(0, N_CHUNKS, chunk_body, None)


@jax.jit
def _run(cond_idx, proc_idx, dx_table, proc_table, visit_table, gamma, beta):
    mesh = plsc.VectorSubcoreMesh(core_axis_name="c", subcore_axis_name="s")
    f = pl.kernel(
        _sc_kernel,
        out_type=[
            jax.ShapeDtypeStruct((N_ROWS, D), jnp.float32),
            jax.ShapeDtypeStruct((N_ROWS, D), jnp.float32),
            jax.ShapeDtypeStruct((B, D), jnp.float32),
        ],
        mesh=mesh,
        scratch_types=[
            pltpu.VMEM((K, GATHER), jnp.int32),
            pltpu.VMEM((CHUNK, D), jnp.float32),
            pltpu.VMEM((D,), jnp.float32),
            pltpu.VMEM((D,), jnp.float32),
            pltpu.VMEM((1, D), jnp.float32),
            pltpu.SemaphoreType.DMA,
        ],
        compiler_params=pltpu.CompilerParams(use_tc_tiling_on_sc=False),
    )
    return f(cond_idx, proc_idx, dx_table, proc_table, visit_table, gamma, beta)


def kernel(conditions_hash, procedures_hash, dx_table, proc_table, visit_table, gamma, beta):
    cond_idx = conditions_hash.reshape(IDX_ROWS, GATHER)
    proc_idx = procedures_hash.reshape(IDX_ROWS, GATHER)
    cond_o, proc_o, visit_o = _run(
        cond_idx, proc_idx, dx_table, proc_table, visit_table, gamma, beta)
    visit_mask = jnp.ones((B, 1), dtype=jnp.float32)
    return (cond_o.reshape(B, L, D), proc_o.reshape(B, L, D),
            visit_o.reshape(B, 1, D), visit_mask)


# speedup vs baseline: 3.5821x; 1.2891x over previous
import jax
import jax.numpy as jnp
from jax import lax
from jax.experimental import pallas as pl
from jax.experimental.pallas import tpu as pltpu
from jax.experimental.pallas import tpu_sc as plsc

VOCAB = 100000
D = 64
B = 4096
L = 200

NC = 2
NS = 16
NW = NC * NS

N_ROWS = B * L
ROWS_PER_W = N_ROWS // NW
CHUNK = 256
GATHER = 128
K = CHUNK // GATHER
N_CHUNKS = ROWS_PER_W // CHUNK
IDX_PER_W = ROWS_PER_W // GATHER
OUT_CHUNK = CHUNK // 2
OUT_PER_W = ROWS_PER_W // 2
VIS_PER_W = (B // 2) // NW


def _rsqrt(v):
    i = lax.bitcast_convert_type(v, jnp.int32)
    i = jnp.int32(0x5F3759DF) - lax.shift_right_logical(i, 1)
    y = lax.bitcast_convert_type(i, jnp.float32)
    h = v * jnp.float32(0.5)
    for _ in range(2):
        y = y * (jnp.float32(1.5) - h * y * y)
    return y


def _hsum(v):
    idx = lax.iota(jnp.int32, 16)
    for k in (1, 2, 4, 8):
        v = v + v.at[idx ^ k].get(mode="promise_in_bounds")
    return v


def _ln_row(xs, gs, bs):
    s = (xs[0] + xs[1]) + (xs[2] + xs[3])
    mean = _hsum(s) * jnp.float32(1.0 / D)
    sq = (xs[0] * xs[0] + xs[1] * xs[1]) + (xs[2] * xs[2] + xs[3] * xs[3])
    ex2 = _hsum(sq) * jnp.float32(1.0 / D)
    var = ex2 - mean * mean
    r = _rsqrt(var + jnp.float32(1e-5))
    return [(xs[j] - mean) * r * gs[j] + bs[j] for j in range(4)]


def _sc_kernel(cond_i, proc_i, dx_t, proc_t, visit_t, gamma, beta,
               cond_o, proc_o, visit_o,
               idx0, idx1, gv0, gv1, ov0, ov1, g_v, b_v, vis_v,
               sg0, sg1, so0, so1):
    idx_v = (idx0, idx1)
    gv = (gv0, gv1)
    ov = (ov0, ov1)
    sg = (sg0, sg1)
    so = (so0, so1)

    wid = lax.axis_index("s") * NC + lax.axis_index("c")

    pltpu.sync_copy(gamma, g_v)
    pltpu.sync_copy(beta, b_v)
    pltpu.sync_copy(visit_t, vis_v)

    gs = [g_v[pl.ds(16 * j, 16)] for j in range(4)]
    bs = [b_v[pl.ds(16 * j, 16)] for j in range(4)]

    vxs = [vis_v[0, pl.ds(16 * j, 16)] for j in range(4)]
    vout = _ln_row(vxs, gs, bs)

    @plsc.parallel_loop(0, VIS_PER_W, unroll=4)
    def _(r):
        for h in range(2):
            for j in range(4):
                ov0[r, pl.ds(16 * (4 * h + j), 16)] = vout[j]

    pltpu.sync_copy(ov0.at[pl.ds(0, VIS_PER_W)],
                    visit_o.at[pl.ds(wid * VIS_PER_W, VIS_PER_W)])

    for idx_hbm, tab, out in ((cond_i, dx_t, cond_o), (proc_i, proc_t, proc_o)):
        idx_base = wid * IDX_PER_W
        out_base = wid * OUT_PER_W

        def fire_chunk(k, p, idx_hbm=idx_hbm, tab=tab):
            pltpu.sync_copy(idx_hbm.at[pl.ds(idx_base + k * K, K)], idx_v[p])
            for j in range(K):
                pltpu.async_copy(tab.at[idx_v[p].at[j]],
                                 gv[p].at[pl.ds(j * GATHER, GATHER)], sg[p])

        def wait_chunk(k, p, tab=tab):
            for j in range(K):
                pltpu.make_async_copy(tab.at[idx_v[p].at[j]],
                                      gv[p].at[pl.ds(j * GATHER, GATHER)],
                                      sg[p]).wait()

        def out_copy(k, p, out=out):
            return pltpu.make_async_copy(
                ov[p], out.at[pl.ds(out_base + k * OUT_CHUNK, OUT_CHUNK)], so[p])

        fire_chunk(0, 0)

        def pair_body(kk, _, fire_chunk=fire_chunk, wait_chunk=wait_chunk,
                      out_copy=out_copy):
            for p in range(2):
                k = 2 * kk + p

                @pl.when(k < N_CHUNKS - 1)
                def _():
                    fire_chunk(k + 1, 1 - p)

                wait_chunk(k, p)

                @pl.when(k >= 2)
                def _():
                    out_copy(k - 2, p).wait()

                @plsc.parallel_loop(0, OUT_CHUNK, unroll=4)
                def _(r):
                    for h in range(2):
                        xs = [gv[p][2 * r + h, pl.ds(16 * j, 16)]
                              for j in range(4)]
                        ys = _ln_row(xs, gs, bs)
                        for j in range(4):
                            ov[p][r, pl.ds(16 * (4 * h + j), 16)] = ys[j]

                out_copy(k, p).start()
            return None

        lax.fori_loop(0, N_CHUNKS // 2, pair_body, None)

        out_copy(N_CHUNKS - 2, 0).wait()
        out_copy(N_CHUNKS - 1, 1).wait()


@jax.jit
def _run(cond_idx, proc_idx, dx_table, proc_table, visit_table, gamma, beta):
    mesh = plsc.VectorSubcoreMesh(core_axis_name="c", subcore_axis_name="s")
    f = pl.kernel(
        _sc_kernel,
        out_type=[
            jax.ShapeDtypeStruct((N_ROWS // 2, 128), jnp.float32),
            jax.ShapeDtypeStruct((N_ROWS // 2, 128), jnp.float32),
            jax.ShapeDtypeStruct((B // 2, 128), jnp.float32),
        ],
        mesh=mesh,
        scratch_types=[
            pltpu.VMEM((K, GATHER), jnp.int32),
            pltpu.VMEM((K, GATHER), jnp.int32),
            pltpu.VMEM((CHUNK, D), jnp.float32),
            pltpu.VMEM((CHUNK, D), jnp.float32),
            pltpu.VMEM((OUT_CHUNK, 128), jnp.float32),
            pltpu.VMEM((OUT_CHUNK, 128), jnp.float32),
            pltpu.VMEM((D,), jnp.float32),
            pltpu.VMEM((D,), jnp.float32),
            pltpu.VMEM((1, D), jnp.float32),
            pltpu.SemaphoreType.DMA,
            pltpu.SemaphoreType.DMA,
            pltpu.SemaphoreType.DMA,
            pltpu.SemaphoreType.DMA,
        ],
        compiler_params=pltpu.CompilerParams(use_tc_tiling_on_sc=False),
    )
    return f(cond_idx, proc_idx, dx_table, proc_table, visit_table, gamma, beta)


def kernel(conditions_hash, procedures_hash, dx_table, proc_table, visit_table, gamma, beta):
    cond_idx = conditions_hash.reshape(N_ROWS // GATHER, GATHER)
    proc_idx = procedures_hash.reshape(N_ROWS // GATHER, GATHER)
    cond_o, proc_o, visit_o = _run(
        cond_idx, proc_idx, dx_table, proc_table, visit_table, gamma, beta)
    visit_mask = jnp.ones((B, 1), dtype=jnp.float32)
    return (cond_o.reshape(B, L, D), proc_o.reshape(B, L, D),
            visit_o.reshape(B, 1, D), visit_mask)


# speedup vs baseline: 4.2172x; 1.1773x over previous
import jax
import jax.numpy as jnp
from jax import lax
from jax.experimental import pallas as pl
from jax.experimental.pallas import tpu as pltpu
from jax.experimental.pallas import tpu_sc as plsc

VOCAB = 100000
D = 64
B = 4096
L = 200

NC = 2
NS = 16
NW = NC * NS

N_ROWS = B * L
ROWS_PER_W = N_ROWS // NW
CHUNK = 512
GATHER = 128
K = CHUNK // GATHER
N_CHUNKS = ROWS_PER_W // CHUNK
VIS_PER_W = B // NW

LN_BLK = 8192


def _ln_tables_body(dx_ref, proc_ref, g_ref, b_ref, dxo_ref, proco_ref):
    g = g_ref[...]
    b = b_ref[...]
    for src, dst in ((dx_ref, dxo_ref), (proc_ref, proco_ref)):
        x = src[...]
        mu = jnp.mean(x, axis=-1, keepdims=True)
        d = x - mu
        var = jnp.mean(d * d, axis=-1, keepdims=True)
        dst[...] = d * lax.rsqrt(var + jnp.float32(1e-5)) * g + b


def _ln_tables(dx_table, proc_table, gamma, beta):
    n = dx_table.shape[0]
    grid = (n + LN_BLK - 1) // LN_BLK
    return pl.pallas_call(
        _ln_tables_body,
        grid=(grid,),
        in_specs=[
            pl.BlockSpec((LN_BLK, D), lambda i: (i, 0)),
            pl.BlockSpec((LN_BLK, D), lambda i: (i, 0)),
            pl.BlockSpec((1, D), lambda i: (0, 0)),
            pl.BlockSpec((1, D), lambda i: (0, 0)),
        ],
        out_specs=[
            pl.BlockSpec((LN_BLK, D), lambda i: (i, 0)),
            pl.BlockSpec((LN_BLK, D), lambda i: (i, 0)),
        ],
        out_shape=[
            jax.ShapeDtypeStruct((n, D), jnp.float32),
            jax.ShapeDtypeStruct((n, D), jnp.float32),
        ],
    )(dx_table, proc_table, gamma.reshape(1, D), beta.reshape(1, D))


def _rsqrt(v):
    i = lax.bitcast_convert_type(v, jnp.int32)
    i = jnp.int32(0x5F3759DF) - lax.shift_right_logical(i, 1)
    y = lax.bitcast_convert_type(i, jnp.float32)
    h = v * jnp.float32(0.5)
    for _ in range(2):
        y = y * (jnp.float32(1.5) - h * y * y)
    return y


def _hsum(v):
    idx = lax.iota(jnp.int32, 16)
    for k in (1, 2, 4, 8):
        v = v + v.at[idx ^ k].get(mode="promise_in_bounds")
    return v


def _ln_row(xs, gs, bs):
    s = (xs[0] + xs[1]) + (xs[2] + xs[3])
    mean = _hsum(s) * jnp.float32(1.0 / D)
    sq = (xs[0] * xs[0] + xs[1] * xs[1]) + (xs[2] * xs[2] + xs[3] * xs[3])
    ex2 = _hsum(sq) * jnp.float32(1.0 / D)
    var = ex2 - mean * mean
    r = _rsqrt(var + jnp.float32(1e-5))
    return [(xs[j] - mean) * r * gs[j] + bs[j] for j in range(4)]


def _sc_kernel(cond_i, proc_i, dx_t, proc_t, visit_t, gamma, beta,
               cond_o, proc_o, visit_o,
               idx0, idx1, gv0, gv1, g_v, b_v, vis_v,
               sg0, sg1, so0, so1):
    idx_v = (idx0, idx1)
    gv = (gv0, gv1)
    sg = (sg0, sg1)
    so = (so0, so1)

    wid = lax.axis_index("s") * NC + lax.axis_index("c")

    pltpu.sync_copy(gamma, g_v)
    pltpu.sync_copy(beta, b_v)
    pltpu.sync_copy(visit_t, vis_v)

    gs = [g_v[pl.ds(16 * j, 16)] for j in range(4)]
    bs = [b_v[pl.ds(16 * j, 16)] for j in range(4)]

    vxs = [vis_v[0, pl.ds(16 * j, 16)] for j in range(4)]
    vout = _ln_row(vxs, gs, bs)

    @plsc.parallel_loop(0, VIS_PER_W, unroll=4)
    def _(r):
        for j in range(4):
            gv0[r, pl.ds(16 * j, 16)] = vout[j]

    pltpu.sync_copy(gv0.at[pl.ds(0, VIS_PER_W)],
                    visit_o.at[pl.ds(wid * VIS_PER_W, VIS_PER_W)])

    for idx_hbm, tab, out in ((cond_i, dx_t, cond_o), (proc_i, proc_t, proc_o)):
        row_base = wid * ROWS_PER_W

        def fire_chunk(k, p, idx_hbm=idx_hbm, tab=tab):
            pltpu.sync_copy(idx_hbm.at[pl.ds(row_base + k * CHUNK, CHUNK)],
                            idx_v[p])
            for j in range(K):
                pltpu.async_copy(tab.at[idx_v[p].at[pl.ds(j * GATHER, GATHER)]],
                                 gv[p].at[pl.ds(j * GATHER, GATHER)], sg[p])

        def wait_chunk(k, p, tab=tab):
            for j in range(K):
                pltpu.make_async_copy(
                    tab.at[idx_v[p].at[pl.ds(j * GATHER, GATHER)]],
                    gv[p].at[pl.ds(j * GATHER, GATHER)], sg[p]).wait()

        def out_copy(k, p, out=out):
            return pltpu.make_async_copy(
                gv[p], out.at[pl.ds(row_base + k * CHUNK, CHUNK)], so[p])

        fire_chunk(0, 0)

        def pair_body(kk, _, fire_chunk=fire_chunk, wait_chunk=wait_chunk,
                      out_copy=out_copy):
            for p in range(2):
                k = 2 * kk + p

                @pl.when(k >= 1)
                def _():
                    out_copy(k - 1, 1 - p).wait()

                @pl.when(k < N_CHUNKS - 1)
                def _():
                    fire_chunk(k + 1, 1 - p)

                wait_chunk(k, p)
                out_copy(k, p).start()
            return None

        lax.fori_loop(0, N_CHUNKS // 2, pair_body, None)

        out_copy(N_CHUNKS - 1, (N_CHUNKS - 1) % 2).wait()


def _gather_pump(cond_idx, proc_idx, dx_n, proc_n, visit_table, gamma, beta):
    mesh = plsc.VectorSubcoreMesh(core_axis_name="c", subcore_axis_name="s")
    f = pl.kernel(
        _sc_kernel,
        out_type=[
            jax.ShapeDtypeStruct((N_ROWS, D), jnp.float32),
            jax.ShapeDtypeStruct((N_ROWS, D), jnp.float32),
            jax.ShapeDtypeStruct((B, D), jnp.float32),
        ],
        mesh=mesh,
        scratch_types=[
            pltpu.VMEM((CHUNK,), jnp.int32),
            pltpu.VMEM((CHUNK,), jnp.int32),
            pltpu.VMEM((CHUNK, D), jnp.float32),
            pltpu.VMEM((CHUNK, D), jnp.float32),
            pltpu.VMEM((D,), jnp.float32),
            pltpu.VMEM((D,), jnp.float32),
            pltpu.VMEM((1, D), jnp.float32),
            pltpu.SemaphoreType.DMA,
            pltpu.SemaphoreType.DMA,
            pltpu.SemaphoreType.DMA,
            pltpu.SemaphoreType.DMA,
        ],
        compiler_params=pltpu.CompilerParams(use_tc_tiling_on_sc=False),
    )
    return f(cond_idx, proc_idx, dx_n, proc_n, visit_table, gamma, beta)


@jax.jit
def _impl(conditions_hash, procedures_hash, dx_table, proc_table, visit_table,
          gamma, beta):
    dx_n, proc_n = _ln_tables(dx_table, proc_table, gamma, beta)
    cond_idx = conditions_hash.reshape(N_ROWS)
    proc_idx = procedures_hash.reshape(N_ROWS)
    cond_o, proc_o, visit_o = _gather_pump(
        cond_idx, proc_idx, dx_n, proc_n, visit_table, gamma, beta)
    visit_mask = jnp.ones((B, 1), dtype=jnp.float32)
    return (cond_o.reshape(B, L, D), proc_o.reshape(B, L, D),
            visit_o.reshape(B, 1, D), visit_mask)


def kernel(conditions_hash, procedures_hash, dx_table, proc_table, visit_table, gamma, beta):
    return _impl(conditions_hash, procedures_hash, dx_table, proc_table,
                 visit_table, gamma, beta)


# speedup vs baseline: 4.4854x; 1.0636x over previous
import jax
import jax.numpy as jnp
from jax import lax
from jax.experimental import pallas as pl
from jax.experimental.pallas import tpu as pltpu
from jax.experimental.pallas import tpu_sc as plsc

VOCAB = 100000
D = 64
B = 4096
L = 200

NC = 2
NS = 16
NW = NC * NS

N_ROWS = B * L
ROWS_PER_W = N_ROWS // NW
CHUNK = 256
GATHER = 128
K = CHUNK // GATHER
N_CHUNKS = ROWS_PER_W // CHUNK
OUT_CHUNK = CHUNK // 2
OUT_PER_W = ROWS_PER_W // 2
VIS_PER_W = (B // 2) // NW

LN_BLK = 8192


def _ln_tables_body(dx_ref, proc_ref, g_ref, b_ref, dxo_ref, proco_ref):
    g = g_ref[...]
    b = b_ref[...]
    for src, dst in ((dx_ref, dxo_ref), (proc_ref, proco_ref)):
        x = src[...]
        mu = jnp.mean(x, axis=-1, keepdims=True)
        d = x - mu
        var = jnp.mean(d * d, axis=-1, keepdims=True)
        dst[...] = d * lax.rsqrt(var + jnp.float32(1e-5)) * g + b


def _ln_tables(dx_table, proc_table, gamma, beta):
    n = dx_table.shape[0]
    grid = (n + LN_BLK - 1) // LN_BLK
    return pl.pallas_call(
        _ln_tables_body,
        grid=(grid,),
        in_specs=[
            pl.BlockSpec((LN_BLK, D), lambda i: (i, 0)),
            pl.BlockSpec((LN_BLK, D), lambda i: (i, 0)),
            pl.BlockSpec((1, D), lambda i: (0, 0)),
            pl.BlockSpec((1, D), lambda i: (0, 0)),
        ],
        out_specs=[
            pl.BlockSpec((LN_BLK, D), lambda i: (i, 0)),
            pl.BlockSpec((LN_BLK, D), lambda i: (i, 0)),
        ],
        out_shape=[
            jax.ShapeDtypeStruct((n, D), jnp.float32),
            jax.ShapeDtypeStruct((n, D), jnp.float32),
        ],
    )(dx_table, proc_table, gamma.reshape(1, D), beta.reshape(1, D))


def _rsqrt(v):
    i = lax.bitcast_convert_type(v, jnp.int32)
    i = jnp.int32(0x5F3759DF) - lax.shift_right_logical(i, 1)
    y = lax.bitcast_convert_type(i, jnp.float32)
    h = v * jnp.float32(0.5)
    for _ in range(2):
        y = y * (jnp.float32(1.5) - h * y * y)
    return y


def _hsum(v):
    idx = lax.iota(jnp.int32, 16)
    for k in (1, 2, 4, 8):
        v = v + v.at[idx ^ k].get(mode="promise_in_bounds")
    return v


def _ln_row(xs, gs, bs):
    s = (xs[0] + xs[1]) + (xs[2] + xs[3])
    mean = _hsum(s) * jnp.float32(1.0 / D)
    sq = (xs[0] * xs[0] + xs[1] * xs[1]) + (xs[2] * xs[2] + xs[3] * xs[3])
    ex2 = _hsum(sq) * jnp.float32(1.0 / D)
    var = ex2 - mean * mean
    r = _rsqrt(var + jnp.float32(1e-5))
    return [(xs[j] - mean) * r * gs[j] + bs[j] for j in range(4)]


def _sc_kernel(do_visit, idx_hbm, tab, visit_t, gamma, beta,
               out, visit_o,
               idx0, idx1, gv0, gv1, ov0, ov1, g_v, b_v, vis_v,
               sg0, sg1, so0, so1):
    idx_v = (idx0, idx1)
    gv = (gv0, gv1)
    ov = (ov0, ov1)
    sg = (sg0, sg1)
    so = (so0, so1)

    wid = lax.axis_index("s") * NC + lax.axis_index("c")

    if do_visit:
        pltpu.sync_copy(gamma, g_v)
        pltpu.sync_copy(beta, b_v)
        pltpu.sync_copy(visit_t, vis_v)

        gs = [g_v[pl.ds(16 * j, 16)] for j in range(4)]
        bs = [b_v[pl.ds(16 * j, 16)] for j in range(4)]
        vxs = [vis_v[0, pl.ds(16 * j, 16)] for j in range(4)]
        vout = _ln_row(vxs, gs, bs)

        @plsc.parallel_loop(0, VIS_PER_W, unroll=4)
        def _(r):
            for h in range(2):
                for j in range(4):
                    ov0[r, pl.ds(16 * (4 * h + j), 16)] = vout[j]

        pltpu.sync_copy(ov0.at[pl.ds(0, VIS_PER_W)],
                        visit_o.at[pl.ds(wid * VIS_PER_W, VIS_PER_W)])

    if True:
        row_base = wid * ROWS_PER_W
        out_base = wid * OUT_PER_W

        def fire_chunk(k, p, idx_hbm=idx_hbm, tab=tab):
            pltpu.sync_copy(idx_hbm.at[pl.ds(row_base + k * CHUNK, CHUNK)],
                            idx_v[p])
            for j in range(K):
                pltpu.async_copy(tab.at[idx_v[p].at[pl.ds(j * GATHER, GATHER)]],
                                 gv[p].at[pl.ds(j * GATHER, GATHER)], sg[p])

        def wait_chunk(k, p, tab=tab):
            for j in range(K):
                pltpu.make_async_copy(
                    tab.at[idx_v[p].at[pl.ds(j * GATHER, GATHER)]],
                    gv[p].at[pl.ds(j * GATHER, GATHER)], sg[p]).wait()

        def out_copy(k, p, out=out):
            return pltpu.make_async_copy(
                ov[p], out.at[pl.ds(out_base + k * OUT_CHUNK, OUT_CHUNK)], so[p])

        fire_chunk(0, 0)

        def pair_body(kk, _, fire_chunk=fire_chunk, wait_chunk=wait_chunk,
                      out_copy=out_copy):
            for p in range(2):
                k = 2 * kk + p

                @pl.when(k < N_CHUNKS - 1)
                def _():
                    fire_chunk(k + 1, 1 - p)

                wait_chunk(k, p)

                @pl.when(k >= 2)
                def _():
                    out_copy(k - 2, p).wait()

                @plsc.parallel_loop(0, OUT_CHUNK, unroll=4)
                def _(r):
                    for h in range(2):
                        for j in range(4):
                            ov[p][r, pl.ds(16 * (4 * h + j), 16)] = (
                                gv[p][2 * r + h, pl.ds(16 * j, 16)])

                out_copy(k, p).start()
            return None

        lax.fori_loop(0, N_CHUNKS // 2, pair_body, None)

        out_copy(N_CHUNKS - 2, 0).wait()
        out_copy(N_CHUNKS - 1, 1).wait()


def _gather_pump(idx_flat, tab_n, visit_table, gamma, beta, do_visit):
    import functools
    mesh = plsc.VectorSubcoreMesh(core_axis_name="c", subcore_axis_name="s")
    f = pl.kernel(
        functools.partial(_sc_kernel, do_visit),
        out_type=[
            jax.ShapeDtypeStruct((N_ROWS // 2, 128), jnp.float32),
            jax.ShapeDtypeStruct((B // 2, 128), jnp.float32),
        ],
        mesh=mesh,
        scratch_types=[
            pltpu.VMEM((CHUNK,), jnp.int32),
            pltpu.VMEM((CHUNK,), jnp.int32),
            pltpu.VMEM((CHUNK, D), jnp.float32),
            pltpu.VMEM((CHUNK, D), jnp.float32),
            pltpu.VMEM((OUT_CHUNK, 128), jnp.float32),
            pltpu.VMEM((OUT_CHUNK, 128), jnp.float32),
            pltpu.VMEM((D,), jnp.float32),
            pltpu.VMEM((D,), jnp.float32),
            pltpu.VMEM((1, D), jnp.float32),
            pltpu.SemaphoreType.DMA,
            pltpu.SemaphoreType.DMA,
            pltpu.SemaphoreType.DMA,
            pltpu.SemaphoreType.DMA,
        ],
        compiler_params=pltpu.CompilerParams(use_tc_tiling_on_sc=False),
    )
    return f(idx_flat, tab_n, visit_table, gamma, beta)


@jax.jit
def _impl(conditions_hash, procedures_hash, dx_table, proc_table, visit_table,
          gamma, beta):
    dx_n, proc_n = _ln_tables(dx_table, proc_table, gamma, beta)
    cond_idx = conditions_hash.reshape(N_ROWS)
    proc_idx = procedures_hash.reshape(N_ROWS)
    cond_p, visit_o = _gather_pump(
        cond_idx, dx_n, visit_table, gamma, beta, True)
    proc_p, _ = _gather_pump(
        proc_idx, proc_n, visit_table, gamma, beta, False)
    cond_o = cond_p
    proc_o = proc_p
    visit_mask = jnp.ones((B, 1), dtype=jnp.float32)
    return (cond_o.reshape(B, L, D), proc_o.reshape(B, L, D),
            visit_o.reshape(B, 1, D), visit_mask)


def kernel(conditions_hash, procedures_hash, dx_table, proc_table, visit_table, gamma, beta):
    return _impl(conditions_hash, procedures_hash, dx_table, proc_table,
                 visit_table, gamma, beta)
